# Initial kernel scaffold; baseline (speedup 1.0000x reference)
import jax
import jax.numpy as jnp
from jax.experimental import pallas as pl


def kernel(hidden_states, expert_indices, W1, b1, W2, b2):
    raise NotImplementedError("write your pallas kernel here")


# speedup vs baseline: 1.3556x; 1.3556x over previous
import functools

import jax
import jax.numpy as jnp
from jax import lax
from jax.experimental import pallas as pl
from jax.experimental.pallas import tpu as pltpu
from jax.experimental.pallas import tpu_sc as plsc

_T = 128
_FC = 1024


def _sc_row_gather(table, idx, n_out):
    rows, d = table.shape
    info = plsc.get_sparse_core_info()
    nc, ns = info.num_cores, info.num_subcores
    nw = nc * ns
    b_per_w = n_out // nw
    mesh = plsc.VectorSubcoreMesh(core_axis_name="c", subcore_axis_name="s")

    @functools.partial(
        pl.kernel,
        mesh=mesh,
        out_type=jax.ShapeDtypeStruct((n_out, d), jnp.float32),
        scratch_types=[
            pltpu.VMEM((b_per_w,), jnp.int32),
            pltpu.VMEM((b_per_w, d), jnp.float32),
            pltpu.SemaphoreType.DMA,
        ],
    )
    def gather(table_hbm, idx_hbm, out_hbm, idx_v, rows_v, sem):
        wid = lax.axis_index("s") * nc + lax.axis_index("c")
        base = wid * b_per_w
        pltpu.sync_copy(idx_hbm.at[pl.ds(base, b_per_w)], idx_v)
        pltpu.async_copy(table_hbm.at[idx_v], rows_v, sem).wait()
        pltpu.sync_copy(rows_v, out_hbm.at[pl.ds(base, b_per_w)])

    return gather(table, idx)


def _mlp_body(te_ref, tv_ref, x_ref, w1_ref, b1_ref, w2_ref, b2_ref, out_ref):
    f = pl.program_id(0)
    t = pl.program_id(1)

    @pl.when(tv_ref[t] == 1)
    def _():
        x = x_ref[...]
        h = jnp.dot(x, w1_ref[0], preferred_element_type=jnp.float32)
        h = jnp.maximum(h + b1_ref[0, 0], 0.0)
        y = jnp.dot(h, w2_ref[0], preferred_element_type=jnp.float32)

        @pl.when(f == 0)
        def _():
            out_ref[pl.ds(t * _T, _T), :] = y + b2_ref[0, 0]

        @pl.when(f != 0)
        def _():
            out_ref[pl.ds(t * _T, _T), :] += y


def _grouped_mlp(x_sorted, tile_expert, tile_valid, W1, b1, W2, b2):
    p, d = x_sorted.shape
    e, _, f_dim = W1.shape
    nt = p // _T
    nf = f_dim // _FC
    b1r = b1.reshape(e, 1, f_dim)
    b2r = b2.reshape(e, 1, d)

    grid_spec = pltpu.PrefetchScalarGridSpec(
        num_scalar_prefetch=2,
        grid=(nf, nt),
        in_specs=[
            pl.BlockSpec((_T, d), lambda f, t, te, tv: (t, 0)),
            pl.BlockSpec((1, d, _FC), lambda f, t, te, tv: (te[t], 0, f)),
            pl.BlockSpec((1, 1, _FC), lambda f, t, te, tv: (te[t], 0, f)),
            pl.BlockSpec((1, _FC, d), lambda f, t, te, tv: (te[t], f, 0)),
            pl.BlockSpec((1, 1, d), lambda f, t, te, tv: (te[t], 0, 0)),
        ],
        out_specs=pl.BlockSpec((p, d), lambda f, t, te, tv: (0, 0)),
    )
    return pl.pallas_call(
        _mlp_body,
        grid_spec=grid_spec,
        out_shape=jax.ShapeDtypeStruct((p, d), jnp.float32),
    )(tile_expert, tile_valid, x_sorted, W1, b1r, W2, b2r)


def _route(flat_indices, n_experts, p_total):
    s = flat_indices.shape[0]
    nt = p_total // _T
    idx = flat_indices.astype(jnp.int32)
    order = jnp.argsort(idx, stable=True).astype(jnp.int32)
    sorted_e = idx[order]
    counts = jnp.bincount(idx, length=n_experts)
    pad_counts = ((counts + _T - 1) // _T) * _T
    zero = jnp.zeros((1,), jnp.int32)
    pad_off = jnp.concatenate([zero, jnp.cumsum(pad_counts).astype(jnp.int32)])
    off = jnp.concatenate([zero, jnp.cumsum(counts).astype(jnp.int32)])
    r = jnp.arange(s, dtype=jnp.int32)
    slot = pad_off[sorted_e] + (r - off[sorted_e])
    src = jnp.zeros((p_total,), jnp.int32).at[slot].set(order)
    pos = jnp.zeros((s,), jnp.int32).at[order].set(slot)
    used = pad_off[n_experts] // _T
    tile_ids = jnp.arange(nt, dtype=jnp.int32)
    te = jnp.searchsorted(pad_off[1:], tile_ids * _T, side="right")
    te = jnp.minimum(te, n_experts - 1).astype(jnp.int32)
    last_e = te[jnp.maximum(used - 1, 0)]
    tile_valid = (tile_ids < used).astype(jnp.int32)
    tile_expert = jnp.where(tile_ids < used, te, last_e).astype(jnp.int32)
    return src, pos, tile_expert, tile_valid


def kernel(hidden_states, expert_indices, W1, b1, W2, b2):
    bsz, seq, d = hidden_states.shape
    e = W1.shape[0]
    s = bsz * seq
    p_total = ((s + e * (_T - 1) + 255) // 256) * 256
    p_total = ((p_total + _T - 1) // _T) * _T

    flat = hidden_states.reshape(s, d)
    src, pos, tile_expert, tile_valid = _route(
        expert_indices.reshape(-1), e, p_total
    )
    x_sorted = _sc_row_gather(flat, src, p_total)
    y_sorted = _grouped_mlp(x_sorted, tile_expert, tile_valid, W1, b1, W2, b2)
    out = _sc_row_gather(y_sorted, pos, s)
    return out.reshape(bsz, seq, d)
